# baseline (speedup 1.0000x reference)
import functools

import jax
import jax.numpy as jnp
from jax import lax
from jax.experimental import pallas as pl
from jax.experimental.pallas import tpu as pltpu
from jax.experimental.pallas import tpu_sc as plsc

_B = 1024
_SRC = 200
_DEC = 512
_ENC = 512
_V = 1000

_L = 16
_NC = 2
_NS = 16
_NW = _NC * _NS
_GPW = _B // _L // _NW
_VB = _L * _V
_SB = _L * _SRC
_VCH = _V // _L
_VPAD = _VB + 128


def _gate_body(dh_ref, cv_ref, w1_ref, b1_ref, w2_ref, b2_ref, p_ref):
    w1 = w1_ref[...]
    h = lax.dot_general(dh_ref[...], w1[:, :_DEC], (((1,), (1,)), ((), ())),
                        preferred_element_type=jnp.float32)
    h += lax.dot_general(cv_ref[...], w1[:, _DEC:], (((1,), (1,)), ((), ())),
                         preferred_element_type=jnp.float32)
    h = jnp.tanh(h + b1_ref[...])
    z = lax.dot_general(h, w2_ref[...], (((1,), (1,)), ((), ())),
                        preferred_element_type=jnp.float32)
    p_ref[...] = jax.nn.sigmoid(z[:, :1] + b2_ref[0, 0])


def _sc_body(attn_hbm, vocab_hbm, chars_hbm, p_hbm, out_hbm,
             vocab_v, out_v, attn_v, chars_v, p_v):
    wid = lax.axis_index("s") * _NC + lax.axis_index("c")
    iota = lax.iota(jnp.int32, _L)
    zeros = jnp.zeros((_L,), jnp.float32)
    vmask = iota < (_V - _VCH * _L)
    soff = iota * _SRC
    voff = iota * _V

    for k in range(_GPW):
        g = wid * _GPW + k
        pltpu.sync_copy(vocab_hbm.at[pl.ds(g * _VB, _VB)],
                        vocab_v.at[pl.ds(0, _VB)])
        pltpu.sync_copy(attn_hbm.at[pl.ds(g * _SB, _SB)], attn_v)
        pltpu.sync_copy(chars_hbm.at[pl.ds(g * _SB, _SB)], chars_v)
        pltpu.sync_copy(p_hbm.at[pl.ds(g * _L, _L)], p_v)

        pv = p_v[...]
        ratio = pv / (1.0 - pv)

        def _scat(s, c):
            idx = soff + s
            ch = plsc.load_gather(chars_v, [idx])
            aw = plsc.load_gather(attn_v, [idx])
            plsc.addupdate_scatter(vocab_v, [voff + ch], aw * ratio)
            return c
        lax.fori_loop(0, _SRC, _scat, 0, unroll=4)

        def _row(r, c):
            pr = plsc.load_gather(p_v, [jnp.full((_L,), r, jnp.int32)])
            one_m_p = 1.0 - pr
            rb = r * _V

            def _ms(i, acc):
                return acc + vocab_v[pl.ds(rb + i * _L, _L)]
            msum_vec = lax.fori_loop(0, _VCH, _ms, zeros, unroll=8)
            tail = vocab_v[pl.ds(rb + _VCH * _L, _L)]
            msum_vec = msum_vec + jnp.where(vmask, tail, 0.0)
            msum = jnp.broadcast_to(jnp.sum(msum_vec), (_L,))
            gs = one_m_p / (one_m_p * msum + 1e-10)

            def _fin(i, c2):
                sl = pl.ds(rb + i * _L, _L)
                out_v[sl] = vocab_v[sl] * gs
                return c2
            lax.fori_loop(0, _VCH + 1, _fin, 0, unroll=8)
            return c
        lax.fori_loop(0, _L, _row, 0)

        pltpu.sync_copy(out_v.at[pl.ds(0, _VB)],
                        out_hbm.at[pl.ds(g * _VB, _VB)])


def kernel(decoder_hidden, context_vector, encoder_outputs, attention_weights,
           vocab_distribution, source_chars, W1, b1, W2, b2):
    del encoder_outputs

    copy_prob = pl.pallas_call(
        _gate_body,
        out_shape=jax.ShapeDtypeStruct((_B, 1), jnp.float32),
    )(decoder_hidden, context_vector, W1,
      b1.reshape(1, _DEC), jnp.pad(W2, ((0, 127), (0, 0))), b2.reshape(1, 1))

    p_flat = copy_prob.reshape(_B)
    chars = source_chars.astype(jnp.int32).reshape(_B * _SRC)

    mesh = plsc.VectorSubcoreMesh(core_axis_name="c", subcore_axis_name="s")
    sc_call = functools.partial(
        pl.kernel, mesh=mesh,
        compiler_params=pltpu.CompilerParams(use_tc_tiling_on_sc=False,
                                             needs_layout_passes=False),
        out_type=jax.ShapeDtypeStruct((_B * _V,), jnp.float32),
        scratch_types=[
            pltpu.VMEM((_VPAD,), jnp.float32),
            pltpu.VMEM((_VPAD,), jnp.float32),
            pltpu.VMEM((_SB,), jnp.float32),
            pltpu.VMEM((_SB,), jnp.int32),
            pltpu.VMEM((_L,), jnp.float32),
        ],
    )(_sc_body)
    final = sc_call(attention_weights.reshape(_B * _SRC),
                    vocab_distribution.reshape(_B * _V), chars, p_flat)
    return final.reshape(_B, _V), copy_prob


# speedup vs baseline: 1.1911x; 1.1911x over previous
import functools

import jax
import jax.numpy as jnp
from jax import lax
from jax.experimental import pallas as pl
from jax.experimental.pallas import tpu as pltpu
from jax.experimental.pallas import tpu_sc as plsc

_B = 1024
_SRC = 200
_DEC = 512
_ENC = 512
_V = 1000

_L = 16
_NC = 2
_NS = 16
_NW = _NC * _NS
_GPW = _B // _L // _NW
_VP = 1024
_SP = 256
_VCH = _VP // _L


def _gate_body(dh_ref, cv_ref, w1_ref, b1_ref, w2_ref, b2_ref, p_ref):
    w1 = w1_ref[...]
    h = lax.dot_general(dh_ref[...], w1[:, :_DEC], (((1,), (1,)), ((), ())),
                        preferred_element_type=jnp.float32)
    h += lax.dot_general(cv_ref[...], w1[:, _DEC:], (((1,), (1,)), ((), ())),
                         preferred_element_type=jnp.float32)
    h = jnp.tanh(h + b1_ref[...])
    z = lax.dot_general(h, w2_ref[...], (((1,), (1,)), ((), ())),
                        preferred_element_type=jnp.float32)
    p_ref[...] = jax.nn.sigmoid(z[:, :1] + b2_ref[0, 0])


def _sc_body(attn_hbm, vocab_hbm, chars_hbm, p_hbm, out_hbm,
             vocab_0, vocab_1, attn_0, attn_1, chars_0, chars_1,
             p_0, p_1, sem_in0, sem_in1, sem_out0, sem_out1):
    wid = lax.axis_index("s") * _NC + lax.axis_index("c")
    iota = lax.iota(jnp.int32, _L)
    zeros = jnp.zeros((_L,), jnp.float32)

    sets = ((vocab_0, attn_0, chars_0, p_0, sem_in0, sem_out0),
            (vocab_1, attn_1, chars_1, p_1, sem_in1, sem_out1))

    in_handles = []
    for k in range(_GPW):
        base = (wid * _GPW + k) * _L
        vocab_v, attn_v, chars_v, p_v, sem_in, _ = sets[k]
        in_handles.append([
            pltpu.async_copy(vocab_hbm.at[pl.ds(base, _L), :], vocab_v,
                             sem_in),
            pltpu.async_copy(attn_hbm.at[pl.ds(base, _L), :], attn_v, sem_in),
            pltpu.async_copy(chars_hbm.at[pl.ds(base, _L), :], chars_v,
                             sem_in),
            pltpu.async_copy(p_hbm.at[pl.ds(base, _L)], p_v, sem_in),
        ])

    out_handles = []
    for k in range(_GPW):
        base = (wid * _GPW + k) * _L
        vocab_v, attn_v, chars_v, p_v, sem_in, sem_out = sets[k]
        for h in in_handles[k]:
            h.wait()

        pv = p_v[...]
        ratio = pv / (1.0 - pv)

        def _scat(s, c, chars_v=chars_v, attn_v=attn_v, vocab_v=vocab_v,
                  ratio=ratio):
            col = jnp.full((_L,), s, jnp.int32)
            ch = plsc.load_gather(chars_v, [iota, col])
            aw = plsc.load_gather(attn_v, [iota, col])
            plsc.addupdate_scatter(vocab_v, [iota, ch], aw * ratio)
            return c
        lax.fori_loop(0, _SRC, _scat, 0, unroll=4)

        def _row(r, c, vocab_v=vocab_v, p_v=p_v):
            r_idx = jnp.full((_L,), r, jnp.int32)
            pr = plsc.load_gather(p_v, [r_idx])
            one_m_p = 1.0 - pr

            def _ms(i, acc):
                return acc + vocab_v[r, pl.ds(i * _L, _L)]
            msum_vec = lax.fori_loop(0, _VCH, _ms, zeros, unroll=8)
            msum = jnp.broadcast_to(jnp.sum(msum_vec), (_L,))
            gs = one_m_p / (one_m_p * msum + 1e-10)

            def _fin(i, c2):
                sl = pl.ds(i * _L, _L)
                vocab_v[r, sl] = vocab_v[r, sl] * gs
                return c2
            lax.fori_loop(0, _VCH, _fin, 0, unroll=8)
            return c
        lax.fori_loop(0, _L, _row, 0)

        out_handles.append(
            pltpu.async_copy(vocab_v, out_hbm.at[pl.ds(base, _L), :],
                             sem_out))

    for h in out_handles:
        h.wait()


def kernel(decoder_hidden, context_vector, encoder_outputs, attention_weights,
           vocab_distribution, source_chars, W1, b1, W2, b2):
    del encoder_outputs

    copy_prob = pl.pallas_call(
        _gate_body,
        out_shape=jax.ShapeDtypeStruct((_B, 1), jnp.float32),
    )(decoder_hidden, context_vector, W1,
      b1.reshape(1, _DEC), jnp.pad(W2, ((0, 127), (0, 0))), b2.reshape(1, 1))

    p_flat = copy_prob.reshape(_B)
    attn_p = jnp.pad(attention_weights, ((0, 0), (0, _SP - _SRC)))
    vocab_p = jnp.pad(vocab_distribution, ((0, 0), (0, _VP - _V)))
    chars_p = jnp.pad(source_chars.astype(jnp.int32),
                      ((0, 0), (0, _SP - _SRC)))

    mesh = plsc.VectorSubcoreMesh(core_axis_name="c", subcore_axis_name="s")
    sc_call = functools.partial(
        pl.kernel, mesh=mesh,
        compiler_params=pltpu.CompilerParams(use_tc_tiling_on_sc=False,
                                             needs_layout_passes=False),
        out_type=jax.ShapeDtypeStruct((_B, _VP), jnp.float32),
        scratch_types=[
            pltpu.VMEM((_L, _VP), jnp.float32),
            pltpu.VMEM((_L, _VP), jnp.float32),
            pltpu.VMEM((_L, _SP), jnp.float32),
            pltpu.VMEM((_L, _SP), jnp.float32),
            pltpu.VMEM((_L, _SP), jnp.int32),
            pltpu.VMEM((_L, _SP), jnp.int32),
            pltpu.VMEM((_L,), jnp.float32),
            pltpu.VMEM((_L,), jnp.float32),
            pltpu.SemaphoreType.DMA,
            pltpu.SemaphoreType.DMA,
            pltpu.SemaphoreType.DMA,
            pltpu.SemaphoreType.DMA,
        ],
    )(_sc_body)
    final_p = sc_call(attn_p, vocab_p, chars_p, p_flat)
    return final_p[:, :_V], copy_prob
